# Initial kernel scaffold; baseline (speedup 1.0000x reference)
import jax
import jax.numpy as jnp
from jax.experimental import pallas as pl


def kernel(prototypes):
    raise NotImplementedError("write your pallas kernel here")


# speedup vs baseline: 1.2805x; 1.2805x over previous
import jax
import jax.numpy as jnp
from jax.experimental import pallas as pl

TOTAL = 81920
DIM = 256
BLOCK = 2048


def _norm_body(x_ref, o_ref):
    x = x_ref[...]
    s = jnp.sum(x * x, axis=-1, keepdims=True)
    o_ref[...] = x * jax.lax.rsqrt(jnp.maximum(s, 1e-24))


def kernel(prototypes):
    return pl.pallas_call(
        _norm_body,
        grid=(TOTAL // BLOCK,),
        in_specs=[pl.BlockSpec((BLOCK, DIM), lambda i: (i, 0))],
        out_specs=pl.BlockSpec((BLOCK, DIM), lambda i: (i, 0)),
        out_shape=jax.ShapeDtypeStruct((TOTAL, DIM), jnp.float32),
    )(prototypes)


# speedup vs baseline: 1.4347x; 1.1204x over previous
import jax
import jax.numpy as jnp
from jax.experimental import pallas as pl

TOTAL = 81920
DIM = 256
BLOCK = 4096


def _norm_body(x_ref, o_ref):
    x = x_ref[...]
    s = jnp.sum(x * x, axis=-1, keepdims=True)
    o_ref[...] = x * jax.lax.rsqrt(jnp.maximum(s, 1e-24))


def kernel(prototypes):
    return pl.pallas_call(
        _norm_body,
        grid=(TOTAL // BLOCK,),
        in_specs=[pl.BlockSpec((BLOCK, DIM), lambda i: (i, 0))],
        out_specs=pl.BlockSpec((BLOCK, DIM), lambda i: (i, 0)),
        out_shape=jax.ShapeDtypeStruct((TOTAL, DIM), jnp.float32),
    )(prototypes)


# speedup vs baseline: 1.4767x; 1.0293x over previous
import jax
import jax.numpy as jnp
from jax.experimental import pallas as pl

TOTAL = 81920
DIM = 256
BLOCK = 8192


def _norm_body(x_ref, o_ref):
    x = x_ref[...]
    s = jnp.sum(x * x, axis=-1, keepdims=True)
    o_ref[...] = x * jax.lax.rsqrt(jnp.maximum(s, 1e-24))


def kernel(prototypes):
    return pl.pallas_call(
        _norm_body,
        grid=(TOTAL // BLOCK,),
        in_specs=[pl.BlockSpec((BLOCK, DIM), lambda i: (i, 0))],
        out_specs=pl.BlockSpec((BLOCK, DIM), lambda i: (i, 0)),
        out_shape=jax.ShapeDtypeStruct((TOTAL, DIM), jnp.float32),
    )(prototypes)


# speedup vs baseline: 1.4966x; 1.0134x over previous
import jax
import jax.numpy as jnp
from jax.experimental import pallas as pl

TOTAL = 81920
DIM = 256
BLOCK = 10240


def _norm_body(x_ref, o_ref):
    x = x_ref[...]
    s = jnp.sum(x * x, axis=-1, keepdims=True)
    o_ref[...] = x * jax.lax.rsqrt(jnp.maximum(s, 1e-24))


def kernel(prototypes):
    return pl.pallas_call(
        _norm_body,
        grid=(TOTAL // BLOCK,),
        in_specs=[pl.BlockSpec((BLOCK, DIM), lambda i: (i, 0))],
        out_specs=pl.BlockSpec((BLOCK, DIM), lambda i: (i, 0)),
        out_shape=jax.ShapeDtypeStruct((TOTAL, DIM), jnp.float32),
    )(prototypes)
